# baseline (speedup 1.0000x reference)
import functools

import jax
import jax.numpy as jnp
from jax import lax
from jax.experimental import pallas as pl
from jax.experimental.pallas import tpu as pltpu
from jax.experimental.pallas import tpu_sc as plsc

NC, NS, LANES = 2, 16, 16
NW = NC * NS
D = 64
SCALE = 8.0
CW = 512


@functools.lru_cache(maxsize=None)
def _build(B: int):
    assert B % (NW * CW) == 0, B
    bpw = B // NW
    nwin = bpw // CW
    mesh = plsc.VectorSubcoreMesh(core_axis_name="c", subcore_axis_name="s")

    @functools.partial(
        pl.kernel,
        out_type=jax.ShapeDtypeStruct((B // 2, 128), jnp.float32),
        mesh=mesh,
        scratch_types=[
            pltpu.VMEM((bpw,), jnp.int32),
            pltpu.VMEM((CW,), jnp.int32),
            pltpu.VMEM((CW, 128), jnp.float32),
            pltpu.VMEM((CW // 2, 128), jnp.float32),
            pltpu.SemaphoreType.DMA,
        ],
    )
    def emb_kernel(x_hbm, emb2_hbm, out_hbm, idx_all, par, rv, cwb, gsem):
        wid = lax.axis_index("s") * NC + lax.axis_index("c")
        base = wid * bpw
        pltpu.sync_copy(x_hbm.at[pl.ds(base, bpw)], idx_all)

        def window(g, carry):
            woff = g * CW
            for j in range(CW // LANES):
                iv = idx_all[pl.ds(woff + j * LANES, LANES)]
                par[pl.ds(j * LANES, LANES)] = lax.bitwise_and(iv, 1)
                hi = lax.shift_right_logical(iv, 1)
                pltpu.async_copy(emb2_hbm.at[hi],
                                 rv.at[pl.ds(j * LANES, LANES)], gsem)
            pltpu.make_async_copy(emb2_hbm.at[pl.ds(0, CW)], rv, gsem).wait()

            def grp(j16, c2):
                i0 = j16 * LANES
                pv = par[pl.ds(i0, LANES)]
                for r in range(LANES):
                    i = i0 + r
                    jr = (i0 // 2) + (r // 2)
                    p = pv[r]
                    for k in range(D // LANES):
                        sl = pl.ds(k * LANES, LANES)
                        sh = pl.ds(D + k * LANES, LANES)
                        a = rv[i, sl]
                        b = rv[i, sh]
                        dst = pl.ds((r % 2) * D + k * LANES, LANES)
                        cwb[jr, dst] = jnp.where(p > 0, b, a) * SCALE
                return c2

            lax.fori_loop(0, CW // LANES, grp, 0)
            off2 = pl.multiple_of((base + woff) // 2, 8)
            pltpu.sync_copy(cwb, out_hbm.at[pl.ds(off2, CW // 2)])
            return carry

        lax.fori_loop(0, nwin, window, 0)

    return emb_kernel


def kernel(x, emb):
    s0, s1 = x.shape
    B = s0 * s1
    xf = x.reshape(-1).astype(jnp.int32)
    emb2 = emb.reshape(-1, 2 * D)
    out = _build(B)(xf, emb2)
    return out.reshape(s0, s1, D)
